# Initial kernel scaffold; baseline (speedup 1.0000x reference)
import jax
import jax.numpy as jnp
from jax.experimental import pallas as pl


def kernel(x_con, x_cat, tables):
    raise NotImplementedError("write your pallas kernel here")


# speedup vs baseline: 1.0174x; 1.0174x over previous
import functools

import jax
import jax.numpy as jnp
from jax import lax
from jax.experimental import pallas as pl
from jax.experimental.pallas import tpu as pltpu
from jax.experimental.pallas import tpu_sc as plsc

N_FIELDS = 26
VOCAB = 100000
EMBED_DIM = 16
BATCH = 16384
D_CON = 13
D_OUT = D_CON + N_FIELDS * EMBED_DIM

NC, NS = 2, 16
NW = NC * NS
B_PER_W = BATCH // NW
CHUNK = 256
N_CHUNK = B_PER_W // CHUNK
SUB = 128
N_SUB = CHUNK // SUB
UNROLL = 8


def _body(x_con_hbm, x_cat_hbm, tables_hbm, out_hbm,
          idx_v, rows_v, out_v, xcon_v, sem):
    wid = lax.axis_index("s") * NC + lax.axis_index("c")
    base = wid * B_PER_W

    def chunk_body(c, carry):
        r0 = base + c * CHUNK

        def field_body(i, carry2):
            off = i * VOCAB
            col = D_CON + i * EMBED_DIM

            for j in range(N_SUB):
                pltpu.sync_copy(
                    x_cat_hbm.at[pl.ds(i * BATCH + r0 + j * SUB, SUB)], idx_v)
                for v in range(SUB // 16):
                    idx_v[pl.ds(v * 16, 16)] = idx_v[pl.ds(v * 16, 16)] + off
                pltpu.async_copy(
                    tables_hbm.at[idx_v],
                    rows_v.at[pl.ds(j * SUB, SUB)],
                    sem).wait()

            def copy_body(r8, carry3):
                r = r8 * UNROLL
                for u in range(UNROLL):
                    out_v[r + u, pl.ds(col, EMBED_DIM)] = rows_v[r + u, :]
                return carry3

            return lax.fori_loop(0, CHUNK // UNROLL, copy_body, carry2)

        lax.fori_loop(0, N_FIELDS, field_body, 0)

        pltpu.sync_copy(x_con_hbm.at[pl.ds(r0 * D_CON, CHUNK * D_CON)],
                        xcon_v.at[pl.ds(0, CHUNK * D_CON)])
        lane = lax.iota(jnp.int32, 16)

        def row_body(r8, carry2):
            r = r8 * UNROLL
            for u in range(UNROLL):
                xv = xcon_v[pl.ds((r + u) * D_CON, 16)]
                cur = out_v[r + u, pl.ds(0, 16)]
                out_v[r + u, pl.ds(0, 16)] = jnp.where(lane < D_CON, xv, cur)
            return carry2

        lax.fori_loop(0, CHUNK // UNROLL, row_body, 0)

        pltpu.sync_copy(out_v, out_hbm.at[pl.ds(r0, CHUNK)])
        return carry

    lax.fori_loop(0, N_CHUNK, chunk_body, 0)


@jax.jit
def _run(x_con_flat, x_cat_flat, tables_flat):
    kern = pl.kernel(
        _body,
        out_type=jax.ShapeDtypeStruct((BATCH, D_OUT), jnp.float32),
        mesh=plsc.VectorSubcoreMesh(core_axis_name="c", subcore_axis_name="s"),
        scratch_types=[
            pltpu.VMEM((SUB,), jnp.int32),
            pltpu.VMEM((CHUNK, EMBED_DIM), jnp.float32),
            pltpu.VMEM((CHUNK, D_OUT), jnp.float32),
            pltpu.VMEM((CHUNK * D_CON + 16,), jnp.float32),
            pltpu.SemaphoreType.DMA,
        ],
        compiler_params=pltpu.CompilerParams(use_tc_tiling_on_sc=False),
    )
    return kern(x_con_flat, x_cat_flat, tables_flat)


def kernel(x_con, x_cat, tables):
    x_con_flat = x_con.reshape(BATCH * D_CON)
    x_cat_flat = x_cat.reshape(N_FIELDS * BATCH)
    tables_flat = tables.reshape(N_FIELDS * VOCAB, EMBED_DIM)
    return _run(x_con_flat, x_cat_flat, tables_flat)


# speedup vs baseline: 1.1107x; 1.0917x over previous
import functools

import jax
import jax.numpy as jnp
from jax import lax
from jax.experimental import pallas as pl
from jax.experimental.pallas import tpu as pltpu
from jax.experimental.pallas import tpu_sc as plsc

N_FIELDS = 26
VOCAB = 100000
EMBED_DIM = 16
BATCH = 16384
D_CON = 13
D_EMB = N_FIELDS * EMBED_DIM
D_OUT = D_CON + D_EMB

NC, NS = 2, 16
NW = NC * NS
B_PER_W = BATCH // NW
CHUNK = 128
N_CHUNK = B_PER_W // CHUNK
UNROLL = 4


def _body(x_con_hbm, x_cat_hbm, tables_hbm, out_hbm,
          idx_all, emb_v, out_v, xcon_v, sem):
    wid = lax.axis_index("s") * NC + lax.axis_index("c")
    base = wid * B_PER_W
    lane = lax.iota(jnp.int32, 16)

    def chunk_body(c, carry):
        r0 = base + c * CHUNK

        pltpu.sync_copy(x_cat_hbm.at[:, pl.ds(r0, CHUNK)], idx_all)

        def rebase_body(i, carry2):
            off = i * VOCAB
            for v in range(CHUNK // 16):
                idx_all[i, pl.ds(v * 16, 16)] = (
                    idx_all[i, pl.ds(v * 16, 16)] + off)
            return carry2

        lax.fori_loop(0, N_FIELDS, rebase_body, 0)

        copies = []
        for i in range(N_FIELDS):
            copies.append(pltpu.async_copy(
                tables_hbm.at[idx_all.at[i]],
                emb_v.at[i],
                sem))
        pltpu.sync_copy(x_con_hbm.at[pl.ds(r0 * D_CON, CHUNK * D_CON)],
                        xcon_v.at[pl.ds(0, CHUNK * D_CON)])
        for cp in copies:
            cp.wait()

        def row_body(r4, carry2):
            for u in range(UNROLL):
                r = r4 * UNROLL + u
                xv = xcon_v[pl.ds(r * D_CON, 16)]
                e0 = emb_v[0, r, :]
                out_v[r, pl.ds(0, 16)] = jnp.where(lane < D_CON, xv, e0)
                for k in range(N_FIELDS):
                    out_v[r, pl.ds(D_CON + k * EMBED_DIM, EMBED_DIM)] = (
                        emb_v[k, r, :])
            return carry2

        lax.fori_loop(0, CHUNK // UNROLL, row_body, 0)

        pltpu.sync_copy(out_v, out_hbm.at[pl.ds(r0, CHUNK)])
        return carry

    lax.fori_loop(0, N_CHUNK, chunk_body, 0)


@jax.jit
def _run(x_con_flat, x_cat, tables_flat):
    kern = pl.kernel(
        _body,
        out_type=jax.ShapeDtypeStruct((BATCH, D_OUT), jnp.float32),
        mesh=plsc.VectorSubcoreMesh(core_axis_name="c", subcore_axis_name="s"),
        scratch_types=[
            pltpu.VMEM((N_FIELDS, CHUNK), jnp.int32),
            pltpu.VMEM((N_FIELDS, CHUNK, EMBED_DIM), jnp.float32),
            pltpu.VMEM((CHUNK, D_OUT), jnp.float32),
            pltpu.VMEM((CHUNK * D_CON + 16,), jnp.float32),
            pltpu.SemaphoreType.DMA,
        ],
        compiler_params=pltpu.CompilerParams(use_tc_tiling_on_sc=False),
    )
    return kern(x_con_flat, x_cat, tables_flat)


def kernel(x_con, x_cat, tables):
    x_con_flat = x_con.reshape(BATCH * D_CON)
    tables_flat = tables.reshape(N_FIELDS * VOCAB, EMBED_DIM)
    return _run(x_con_flat, x_cat, tables_flat)


# speedup vs baseline: 2.1542x; 1.9394x over previous
import functools

import jax
import jax.numpy as jnp
from jax import lax
from jax.experimental import pallas as pl
from jax.experimental.pallas import tpu as pltpu
from jax.experimental.pallas import tpu_sc as plsc

N_FIELDS = 26
VOCAB = 100000
EMBED_DIM = 16
BATCH = 16384
D_CON = 13
D_OUT = D_CON + N_FIELDS * EMBED_DIM

NC, NS, L = 2, 16, 16
HALF = BATCH // NC
BW = HALF // NS
STRIPE = 6256
V0_CAP = VOCAB - STRIPE
GSUB = 128
N_G = EMBED_DIM * BW // GSUB


def _body(x_con_hbm, x_cat_hbm, tables_hbm, out_hbm,
          idx_v, fidx_v, outb_v, slab_sh,
          slab_sem, idx_sem, g_sem):
    sc = lax.axis_index("c")
    t = lax.axis_index("s")
    col0 = sc * HALF + t * BW
    stripe_v0 = pl.multiple_of(jnp.minimum(t * STRIPE, V0_CAP), 8)

    pltpu.sync_copy(x_con_hbm.at[:, pl.ds(col0, BW)],
                    outb_v.at[pl.ds(0, D_CON)])
    pltpu.sync_copy(outb_v.at[pl.ds(0, D_CON)],
                    out_hbm.at[pl.ds(0, D_CON), pl.ds(col0, BW)])

    def stage_slab(i):
        for e in range(EMBED_DIM):
            pltpu.async_copy(
                tables_hbm.at[i, e, pl.ds(stripe_v0, STRIPE)],
                slab_sh.at[pl.ds(e * VOCAB + stripe_v0, STRIPE)],
                slab_sem)

    def wait_slab():
        for e in range(EMBED_DIM):
            pltpu.make_async_copy(
                tables_hbm.at[0, 0, pl.ds(0, STRIPE)],
                slab_sh.at[pl.ds(0, STRIPE)], slab_sem).wait()

    def fetch_idx(i):
        pltpu.async_copy(
            x_cat_hbm.at[pl.ds(i * BATCH + col0, BW)], idx_v, idx_sem)

    def wait_idx():
        pltpu.make_async_copy(
            x_cat_hbm.at[pl.ds(0, BW)], idx_v, idx_sem).wait()

    stage_slab(0)
    fetch_idx(0)

    def do_field(i, carry):
        wait_idx()

        def build(j, c2):
            v16 = idx_v[pl.ds(j * L, L)]
            for e in range(EMBED_DIM):
                fidx_v[pl.ds(e * BW + j * L, L)] = v16 + e * VOCAB
            return c2

        lax.fori_loop(0, BW // L, build, 0)

        wait_slab()
        plsc.subcore_barrier()

        copies = []
        for c in range(N_G):
            copies.append(pltpu.async_copy(
                slab_sh.at[fidx_v.at[pl.ds(c * GSUB, GSUB)]],
                outb_v.at[c // (BW // GSUB),
                          pl.ds((c % (BW // GSUB)) * GSUB, GSUB)],
                g_sem))
        fetch_idx(jnp.minimum(i + 1, N_FIELDS - 1))
        for cp in copies:
            cp.wait()

        pltpu.sync_copy(
            outb_v, out_hbm.at[pl.ds(D_CON + i * EMBED_DIM, EMBED_DIM),
                               pl.ds(col0, BW)])

        plsc.subcore_barrier()
        stage_slab(jnp.minimum(i + 1, N_FIELDS - 1))
        return carry

    lax.fori_loop(0, N_FIELDS, do_field, 0)

    wait_idx()
    wait_slab()


@jax.jit
def _run(x_con_t, x_cat_flat, tables_t):
    kern = pl.kernel(
        _body,
        out_type=jax.ShapeDtypeStruct((D_OUT, BATCH), jnp.float32),
        mesh=plsc.VectorSubcoreMesh(core_axis_name="c", subcore_axis_name="s"),
        scratch_types=[
            pltpu.VMEM((BW,), jnp.int32),
            pltpu.VMEM((EMBED_DIM * BW,), jnp.int32),
            pltpu.VMEM((EMBED_DIM, BW), jnp.float32),
            pltpu.VMEM_SHARED((EMBED_DIM * VOCAB,), jnp.float32),
            pltpu.SemaphoreType.DMA,
            pltpu.SemaphoreType.DMA,
            pltpu.SemaphoreType.DMA,
        ],
        compiler_params=pltpu.CompilerParams(use_tc_tiling_on_sc=False),
    )
    return kern(x_con_t, x_cat_flat, tables_t)


def kernel(x_con, x_cat, tables):
    x_con_t = x_con.T
    x_cat_flat = x_cat.reshape(N_FIELDS * BATCH)
    tables_t = tables.transpose(0, 2, 1)
    return _run(x_con_t, x_cat_flat, tables_t).T


# speedup vs baseline: 3.1712x; 1.4721x over previous
import functools

import jax
import jax.numpy as jnp
from jax import lax
from jax.experimental import pallas as pl
from jax.experimental.pallas import tpu as pltpu
from jax.experimental.pallas import tpu_sc as plsc

N_FIELDS = 26
VOCAB = 100000
EMBED_DIM = 16
BATCH = 16384
D_CON = 13
D_OUT = D_CON + N_FIELDS * EMBED_DIM

NC, NS, L = 2, 16, 16
F_PER_SC = N_FIELDS // NC
BW = BATCH // NS
STRIPE = 6256
V0_CAP = VOCAB - STRIPE
GSUB = 128
EQ = 4
N_Q = EMBED_DIM // EQ


def _body(x_con_hbm, x_cat_hbm, tables_hbm, out_hbm,
          idx_v, fidx_v, outb_v, slab_a, slab_b,
          sem_a, sem_b, idx_sem, g_sem):
    sc = lax.axis_index("c")
    t = lax.axis_index("s")
    col0 = t * BW
    stripe_v0 = pl.multiple_of(jnp.minimum(t * STRIPE, V0_CAP), 8)
    f0 = sc * F_PER_SC

    pltpu.sync_copy(x_con_hbm.at[:, pl.ds(col0, BW)],
                    outb_v.at[pl.ds(0, D_CON)])
    pltpu.sync_copy(outb_v.at[pl.ds(0, D_CON)],
                    out_hbm.at[pl.ds(0, D_CON), pl.ds(col0, BW)])

    def stage_quarter(i, q, slot, sem):
        for e in range(EQ):
            pltpu.async_copy(
                tables_hbm.at[i, q * EQ + e, pl.ds(stripe_v0, STRIPE)],
                slot.at[pl.ds(e * VOCAB + stripe_v0, STRIPE)],
                sem)

    def wait_quarter(slot, sem):
        for e in range(EQ):
            pltpu.make_async_copy(
                tables_hbm.at[0, 0, pl.ds(0, STRIPE)],
                slot.at[pl.ds(0, STRIPE)], sem).wait()

    def fetch_idx(i):
        pltpu.async_copy(
            x_cat_hbm.at[pl.ds(i * BATCH + col0, BW)], idx_v, idx_sem)

    def wait_idx():
        pltpu.make_async_copy(
            x_cat_hbm.at[pl.ds(0, BW)], idx_v, idx_sem).wait()

    def gather_quarter(slot, q):
        copies = []
        for c in range(EQ * BW // GSUB):
            e = c // (BW // GSUB)
            cc = c % (BW // GSUB)
            copies.append(pltpu.async_copy(
                slot.at[fidx_v.at[pl.ds(e * BW + cc * GSUB, GSUB)]],
                outb_v.at[q * EQ + e, pl.ds(cc * GSUB, GSUB)],
                g_sem))
        for cp in copies:
            cp.wait()

    stage_quarter(f0, 0, slab_a, sem_a)
    stage_quarter(f0, 1, slab_b, sem_b)
    fetch_idx(f0)

    def do_field(j, carry):
        i = f0 + j
        nxt = jnp.minimum(i + 1, f0 + F_PER_SC - 1)
        wait_idx()

        def build(w, c2):
            v16 = idx_v[pl.ds(w * L, L)]
            for e in range(EQ):
                fidx_v[pl.ds(e * BW + w * L, L)] = v16 + e * VOCAB
            return c2

        lax.fori_loop(0, BW // L, build, 0)
        fetch_idx(nxt)

        for q in range(N_Q):
            slot = slab_a if q % 2 == 0 else slab_b
            sem = sem_a if q % 2 == 0 else sem_b
            wait_quarter(slot, sem)
            plsc.subcore_barrier()
            gather_quarter(slot, q)
            plsc.subcore_barrier()
            if q < N_Q - 2:
                stage_quarter(i, q + 2, slot, sem)
            else:
                stage_quarter(nxt, q + 2 - N_Q, slot, sem)

        pltpu.sync_copy(
            outb_v, out_hbm.at[pl.ds(D_CON + i * EMBED_DIM, EMBED_DIM),
                               pl.ds(col0, BW)])
        return carry

    lax.fori_loop(0, F_PER_SC, do_field, 0)

    wait_idx()
    wait_quarter(slab_a, sem_a)
    wait_quarter(slab_b, sem_b)


@jax.jit
def _run(x_con_t, x_cat_flat, tables_t):
    kern = pl.kernel(
        _body,
        out_type=jax.ShapeDtypeStruct((D_OUT, BATCH), jnp.float32),
        mesh=plsc.VectorSubcoreMesh(core_axis_name="c", subcore_axis_name="s"),
        scratch_types=[
            pltpu.VMEM((BW,), jnp.int32),
            pltpu.VMEM((EQ * BW,), jnp.int32),
            pltpu.VMEM((EMBED_DIM, BW), jnp.float32),
            pltpu.VMEM_SHARED((EQ * VOCAB,), jnp.float32),
            pltpu.VMEM_SHARED((EQ * VOCAB,), jnp.float32),
            pltpu.SemaphoreType.DMA,
            pltpu.SemaphoreType.DMA,
            pltpu.SemaphoreType.DMA,
            pltpu.SemaphoreType.DMA,
        ],
        compiler_params=pltpu.CompilerParams(use_tc_tiling_on_sc=False),
    )
    return kern(x_con_t, x_cat_flat, tables_t)


def kernel(x_con, x_cat, tables):
    x_con_t = x_con.T
    x_cat_flat = x_cat.reshape(N_FIELDS * BATCH)
    tables_t = tables.transpose(0, 2, 1)
    return _run(x_con_t, x_cat_flat, tables_t).T
